# baseline (speedup 1.0000x reference)
import functools

import jax
import jax.numpy as jnp
from jax import lax
from jax.experimental import pallas as pl
from jax.experimental.pallas import tpu as pltpu
from jax.experimental.pallas import tpu_sc as plsc

EMB_DIM = 1024
NUM_CORES = 2
NUM_SUBCORES = 16
NUM_WORKERS = NUM_CORES * NUM_SUBCORES
CHUNK = 64


@functools.partial(jax.jit, static_argnames=())
def _lookup(seg_flat, table):
    n = seg_flat.shape[0]
    per_w = n // NUM_WORKERS
    n_chunks = per_w // CHUNK
    mesh = plsc.VectorSubcoreMesh(core_axis_name="c", subcore_axis_name="s")

    @functools.partial(
        pl.kernel,
        out_type=jax.ShapeDtypeStruct((n, EMB_DIM), jnp.float32),
        mesh=mesh,
        scratch_types=[
            pltpu.VMEM((per_w,), jnp.int32),
            pltpu.VMEM((CHUNK, EMB_DIM), jnp.float32),
            pltpu.SemaphoreType.DMA,
        ],
    )
    def body(seg_hbm, table_hbm, out_hbm, idx_v, rows_v, sem):
        wid = lax.axis_index("s") * NUM_CORES + lax.axis_index("c")
        base = wid * per_w
        pltpu.sync_copy(seg_hbm.at[pl.ds(base, per_w)], idx_v)

        def chunk_body(j, carry):
            off = j * CHUNK
            pltpu.async_copy(
                table_hbm.at[idx_v.at[pl.ds(off, CHUNK)]], rows_v, sem
            ).wait()
            pltpu.sync_copy(rows_v, out_hbm.at[pl.ds(base + off, CHUNK)])
            return carry

        lax.fori_loop(0, n_chunks, chunk_body, 0)

    return body(seg_flat, table)


def kernel(segment, table):
    b, s = segment.shape
    seg_flat = segment.reshape(b * s).astype(jnp.int32)
    out = _lookup(seg_flat, table)
    return out.reshape(b, s, EMB_DIM)


# speedup vs baseline: 2.7813x; 2.7813x over previous
import functools

import jax
import jax.numpy as jnp
from jax import lax
from jax.experimental import pallas as pl
from jax.experimental.pallas import tpu as pltpu
from jax.experimental.pallas import tpu_sc as plsc

EMB_DIM = 1024
LANES = 16
NUM_CORES = 2
NUM_SUBCORES = 16
NUM_WORKERS = NUM_CORES * NUM_SUBCORES
CHUNK = 32


@jax.jit
def _lookup(seg_flat, table):
    n = seg_flat.shape[0]
    per_w = n // NUM_WORKERS
    n_chunks = per_w // CHUNK
    mesh = plsc.VectorSubcoreMesh(core_axis_name="c", subcore_axis_name="s")

    @functools.partial(
        pl.kernel,
        out_type=jax.ShapeDtypeStruct((n, EMB_DIM), jnp.float32),
        mesh=mesh,
        scratch_types=[
            pltpu.VMEM((per_w,), jnp.int32),
            pltpu.VMEM((3, EMB_DIM), jnp.float32),
            pltpu.VMEM((2, CHUNK, EMB_DIM), jnp.float32),
            pltpu.SemaphoreType.DMA((2,)),
        ],
    )
    def body(seg_hbm, table_hbm, out_hbm, idx_v, table_v, rows_v, w_sem):
        wid = lax.axis_index("s") * NUM_CORES + lax.axis_index("c")
        base = wid * per_w
        pltpu.sync_copy(table_hbm, table_v)
        pltpu.sync_copy(seg_hbm.at[pl.ds(base, per_w)], idx_v)

        def step(j, carry):
            buf = lax.rem(j, 2)

            @pl.when(j >= 2)
            def _():
                pltpu.make_async_copy(
                    rows_v.at[buf],
                    out_hbm.at[pl.ds(base, CHUNK)],
                    w_sem.at[buf],
                ).wait()

            for g in range(CHUNK // LANES):
                seg_vec = idx_v[pl.ds(j * CHUNK + g * LANES, LANES)]
                for r in range(LANES):
                    seg = seg_vec[r]
                    for cb in range(0, EMB_DIM // LANES, 8):
                        vals = [
                            table_v[seg, pl.ds((cb + k) * LANES, LANES)]
                            for k in range(8)
                        ]
                        for k in range(8):
                            rows_v[
                                buf, g * LANES + r, pl.ds((cb + k) * LANES, LANES)
                            ] = vals[k]

            pltpu.async_copy(
                rows_v.at[buf],
                out_hbm.at[pl.ds(base + j * CHUNK, CHUNK)],
                w_sem.at[buf],
            )
            return carry

        lax.fori_loop(0, n_chunks, step, 0)

        pltpu.make_async_copy(
            rows_v.at[0], out_hbm.at[pl.ds(base, CHUNK)], w_sem.at[0]
        ).wait()
        pltpu.make_async_copy(
            rows_v.at[1], out_hbm.at[pl.ds(base, CHUNK)], w_sem.at[1]
        ).wait()

    return body(seg_flat, table)


def kernel(segment, table):
    b, s = segment.shape
    seg_flat = segment.reshape(b * s).astype(jnp.int32)
    out = _lookup(seg_flat, table)
    return out.reshape(b, s, EMB_DIM)


# speedup vs baseline: 10.3749x; 3.7303x over previous
import functools

import jax
import jax.numpy as jnp
from jax import lax
from jax.experimental import pallas as pl
from jax.experimental.pallas import tpu as pltpu
from jax.experimental.pallas import tpu_sc as plsc

EMB_DIM = 1024
LANES = 16
NUM_CORES = 2
NUM_SUBCORES = 16
NUM_WORKERS = NUM_CORES * NUM_SUBCORES


@jax.jit
def _lookup(seg_flat, table):
    n = seg_flat.shape[0]
    per_w = n // NUM_WORKERS
    n_groups = per_w // LANES
    mesh = plsc.VectorSubcoreMesh(core_axis_name="c", subcore_axis_name="s")

    @functools.partial(
        pl.kernel,
        out_type=jax.ShapeDtypeStruct((n, EMB_DIM), jnp.float32),
        mesh=mesh,
        scratch_types=[
            pltpu.VMEM((per_w,), jnp.int32),
            pltpu.VMEM((3, EMB_DIM), jnp.float32),
            pltpu.SemaphoreType.DMA,
        ],
    )
    def body(seg_hbm, table_hbm, out_hbm, idx_v, table_v, sem):
        wid = lax.axis_index("s") * NUM_CORES + lax.axis_index("c")
        base = wid * per_w
        pltpu.sync_copy(table_hbm, table_v)
        pltpu.sync_copy(seg_hbm.at[pl.ds(base, per_w)], idx_v)

        def group(g, carry):
            seg_vec = idx_v[pl.ds(g * LANES, LANES)]
            tok = base + g * LANES
            for r in range(LANES):
                pltpu.async_copy(
                    table_v.at[seg_vec[r]], out_hbm.at[tok + r], sem
                )
            return carry

        lax.fori_loop(0, n_groups, group, 0)

        pltpu.make_async_copy(
            out_hbm.at[pl.ds(base, per_w)],
            out_hbm.at[pl.ds(base, per_w)],
            sem,
        ).wait()

    return body(seg_flat, table)


def kernel(segment, table):
    b, s = segment.shape
    seg_flat = segment.reshape(b * s).astype(jnp.int32)
    out = _lookup(seg_flat, table)
    return out.reshape(b, s, EMB_DIM)
